# baseline (speedup 1.0000x reference)
import functools

import numpy as np
import jax
import jax.numpy as jnp
from jax import lax
from jax.experimental import pallas as pl
from jax.experimental.pallas import tpu as pltpu
from jax.experimental.pallas import tpu_sc as plsc

_KWIN = 25
_S = 1024
_B = 2
_HW = 64
_N = _HW * _HW
_BN = _B * _N
_SCALE = 1.0 / np.sqrt(32.0)


def _local_indices_np(H, W, K):
    r = K // 2
    d = np.arange(-r, r + 1)
    ci = np.clip(np.arange(H)[:, None] + d[None, :], 0, H - 1)
    cj = np.clip(np.arange(W)[:, None] + d[None, :], 0, W - 1)
    idx = (ci[:, None, :, None] * W + cj[None, :, None, :]).reshape(H * W, K * K)
    return idx.astype(np.int32)


@functools.lru_cache(maxsize=1)
def _sample_inds_packed():
    local = np.broadcast_to(_local_indices_np(_HW, _HW, _KWIN)[None],
                            (_B, _N, _KWIN * _KWIN))
    n_rand = _S - _KWIN * _KWIN
    with jax.ensure_compile_time_eval():
        rnd = np.asarray(
            jax.random.randint(jax.random.key(42), (_B, _N, n_rand),
                               0, _N, dtype=jnp.int32))
    inds = np.concatenate([local, rnd], axis=-1).astype(np.int32)
    iw = inds.reshape(_BN, _S // 32, 16, 2)
    words = iw[..., 0] | (iw[..., 1] << 16)
    return jnp.asarray(words.reshape(_BN, _S // 2))


def _norm_body(x_ref, o_ref):
    x = x_ref[...]
    row = lax.broadcasted_iota(jnp.int32, x.shape, 0) % 3
    mean = jnp.where(row == 0, 123.675, jnp.where(row == 1, 116.28, 103.53))
    inv = jnp.where(row == 0, 1.0 / 58.395,
                    jnp.where(row == 1, 1.0 / 57.12, 1.0 / 57.375))
    o_ref[...] = (x - mean) * inv


def _normalize(img):
    x = img.reshape(_B * 3, 256 * 256)
    out = pl.pallas_call(
        _norm_body,
        out_shape=jax.ShapeDtypeStruct(x.shape, jnp.float32),
    )(x)
    return out.reshape(_B, 3, 256, 256)


def _conv1_body(x_ref, w_ref, o_ref):
    o_ref[...] = jax.nn.relu(
        lax.dot_general(x_ref[...], w_ref[...], (((0,), (0,)), ((), ())),
                        preferred_element_type=jnp.float32)).astype(jnp.bfloat16)


def _conv1(patches, w1):
    bm = 4096
    m = patches.shape[1]
    return pl.pallas_call(
        _conv1_body,
        grid=(m // bm,),
        in_specs=[
            pl.BlockSpec((27, bm), lambda i: (0, i)),
            pl.BlockSpec((27, 64), lambda i: (0, 0)),
        ],
        out_specs=pl.BlockSpec((bm, 64), lambda i: (i, 0)),
        out_shape=jax.ShapeDtypeStruct((m, 64), jnp.bfloat16),
    )(patches, w1)


def _heads_body(x_ref, w2_ref, wq1_ref, wq2_ref, wk1_ref, wk2_ref,
                q_ref, k_ref):
    acc = jnp.dot(x_ref[0], w2_ref[0], preferred_element_type=jnp.float32)
    for t in range(1, 9):
        acc += jnp.dot(x_ref[t], w2_ref[t], preferred_element_type=jnp.float32)
    f2 = jax.nn.relu(acc)
    qh = jax.nn.relu(
        jnp.dot(f2, wq1_ref[...], preferred_element_type=jnp.float32))
    q_ref[...] = jnp.dot(qh, wq2_ref[...],
                         preferred_element_type=jnp.float32).astype(jnp.bfloat16)
    kh = jax.nn.relu(
        jnp.dot(f2, wk1_ref[...], preferred_element_type=jnp.float32))
    k_ref[...] = jnp.dot(kh, wk2_ref[...],
                         preferred_element_type=jnp.float32).astype(jnp.bfloat16)


def _heads(p2, w2s, wq1m, wq2m, wk1m, wk2m):
    bm = 512
    qf, kf = pl.pallas_call(
        _heads_body,
        grid=(_BN // bm,),
        in_specs=[
            pl.BlockSpec((9, bm, 64), lambda i: (0, i, 0)),
            pl.BlockSpec((9, 64, 256), lambda i: (0, 0, 0)),
            pl.BlockSpec((256, 64), lambda i: (0, 0)),
            pl.BlockSpec((64, 32), lambda i: (0, 0)),
            pl.BlockSpec((256, 64), lambda i: (0, 0)),
            pl.BlockSpec((64, 32), lambda i: (0, 0)),
        ],
        out_specs=[
            pl.BlockSpec((bm, 32), lambda i: (i, 0)),
            pl.BlockSpec((bm, 32), lambda i: (i, 0)),
        ],
        out_shape=[
            jax.ShapeDtypeStruct((_BN, 32), jnp.bfloat16),
            jax.ShapeDtypeStruct((_BN, 32), jnp.bfloat16),
        ],
    )(p2, w2s, wq1m, wq2m, wk1m, wk2m)
    return qf, kf


def _scores_body(q_ref, kt_ref, o_ref):
    s = jnp.dot(q_ref[0], kt_ref[0], preferred_element_type=jnp.float32)
    o_ref[0] = (s * _SCALE).astype(jnp.bfloat16)


def _scores(qf3, kT):
    bm = 256
    return pl.pallas_call(
        _scores_body,
        grid=(_B, _N // bm),
        in_specs=[
            pl.BlockSpec((1, bm, 32), lambda b, i: (b, i, 0)),
            pl.BlockSpec((1, 32, _N), lambda b, i: (b, 0, 0)),
        ],
        out_specs=pl.BlockSpec((1, bm, _N), lambda b, i: (b, i, 0)),
        out_shape=jax.ShapeDtypeStruct((_B, _N, _N), jnp.bfloat16),
    )(qf3, kT)


def _sc_gather(scoresw, indsw, seg):
    NW = 32
    RPW = _BN // NW
    NCH = _S // 32
    mesh = plsc.VectorSubcoreMesh(core_axis_name="c", subcore_axis_name="s")

    @functools.partial(
        pl.kernel,
        out_type=[
            jax.ShapeDtypeStruct((_BN, _S), jnp.float32),
            jax.ShapeDtypeStruct((_BN, _S // 2), jnp.int32),
        ],
        mesh=mesh,
        compiler_params=pltpu.CompilerParams(needs_layout_passes=False),
        scratch_types=[
            pltpu.VMEM((_N // 2,), jnp.int32),
            pltpu.VMEM((_N // 2,), jnp.int32),
            pltpu.VMEM((_S // 2,), jnp.int32),
            pltpu.VMEM((_S // 2,), jnp.int32),
            pltpu.VMEM((_S,), jnp.float32),
            pltpu.VMEM((_S,), jnp.float32),
            pltpu.VMEM((_S // 2,), jnp.int32),
            pltpu.VMEM((_S // 2,), jnp.int32),
            pltpu.VMEM((_N,), jnp.int32),
        ] + [pltpu.SemaphoreType.DMA] * 8,
    )
    def k(scores_hbm, inds_hbm, seg_hbm, lg_hbm, sm_hbm,
          srow0, srow1, idx0, idx1, out0, out1, samp0, samp1, segv,
          si0, si1, ii0, ii1, lo0, lo1, so0, so1):
        srows, idxs, outs, samps = (srow0, srow1), (idx0, idx1), (out0, out1), (samp0, samp1)
        sis, iis, los, sos = (si0, si1), (ii0, ii1), (lo0, lo1), (so0, so1)
        wid = lax.axis_index("s") * 2 + lax.axis_index("c")
        base = wid * RPW
        b = wid // (NW // _B)
        pltpu.sync_copy(seg_hbm.at[b], segv)
        pltpu.async_copy(scores_hbm.at[base], srow0, si0)
        pltpu.async_copy(inds_hbm.at[base], idx0, ii0)
        iota = lax.iota(jnp.int32, 16)

        def pair_body(pr, carry):
            for s in range(2):
                r = pr * 2 + s
                row = base + r

                @pl.when(r + 1 < RPW)
                def _prefetch():
                    pltpu.async_copy(scores_hbm.at[row + 1], srows[1 - s],
                                     sis[1 - s])
                    pltpu.async_copy(inds_hbm.at[row + 1], idxs[1 - s],
                                     iis[1 - s])

                pltpu.make_async_copy(scores_hbm.at[row], srows[s], sis[s]).wait()
                pltpu.make_async_copy(inds_hbm.at[row], idxs[s], iis[s]).wait()

                @pl.when(r >= 2)
                def _drain():
                    pltpu.make_async_copy(outs[s], lg_hbm.at[row - 2],
                                          los[s]).wait()
                    pltpu.make_async_copy(samps[s], sm_hbm.at[row - 2],
                                          sos[s]).wait()

                for c in range(NCH):
                    w = idxs[s][pl.ds(c * 16, 16)]
                    ia = jnp.bitwise_and(w, 0xFFFF)
                    ib = lax.shift_right_logical(w, 16)
                    wa = plsc.load_gather(srows[s],
                                          [lax.shift_right_logical(ia, 1)])
                    wb = plsc.load_gather(srows[s],
                                          [lax.shift_right_logical(ib, 1)])
                    sha = lax.shift_left(jnp.bitwise_and(ia, 1), 4)
                    shb = lax.shift_left(jnp.bitwise_and(ib, 1), 4)
                    va = plsc.bitcast(
                        lax.shift_left(lax.shift_right_logical(wa, sha), 16),
                        jnp.float32)
                    vb = plsc.bitcast(
                        lax.shift_left(lax.shift_right_logical(wb, shb), 16),
                        jnp.float32)
                    posa = c * 32 + iota * 2
                    plsc.store_scatter(outs[s], [posa], va)
                    plsc.store_scatter(outs[s], [posa + 1], vb)
                    sa = plsc.load_gather(segv, [ia])
                    sb = plsc.load_gather(segv, [ib])
                    samps[s][pl.ds(c * 16, 16)] = plsc.bitcast(
                        plsc.pack(sa, sb, format=plsc.PackFormat.INTERLEAVED),
                        jnp.int32)

                pltpu.async_copy(outs[s], lg_hbm.at[row], los[s])
                pltpu.async_copy(samps[s], sm_hbm.at[row], sos[s])
            return carry

        lax.fori_loop(0, RPW // 2, pair_body, 0)
        pltpu.make_async_copy(out0, lg_hbm.at[base + RPW - 2], lo0).wait()
        pltpu.make_async_copy(samp0, sm_hbm.at[base + RPW - 2], so0).wait()
        pltpu.make_async_copy(out1, lg_hbm.at[base + RPW - 1], lo1).wait()
        pltpu.make_async_copy(samp1, sm_hbm.at[base + RPW - 1], so1).wait()

    return k(scoresw, indsw, seg)


def _loss_body(l_ref, s_ref, g_ref, loss_ref, acc_ref):
    i = pl.program_id(0)

    @pl.when(i == 0)
    def _init():
        acc_ref[0] = 0.0
        acc_ref[1] = 0.0

    l = l_ref[...]
    m = jnp.max(l, axis=1, keepdims=True)
    ex = jnp.exp(l - m)
    den = jnp.sum(ex, axis=1, keepdims=True)
    y_pred = ex / (den + 1e-12)
    ylog = jnp.log(jnp.maximum(y_pred, 1e-8))
    seg = g_ref[...]
    t = (s_ref[...].astype(jnp.int32) == seg).astype(jnp.float32)
    tsum = jnp.sum(t, axis=1, keepdims=True)
    y_true = t / (tsum + 1e-9)
    lyt = jnp.log(jnp.maximum(y_true, 1e-30))
    kl_el = jnp.where(y_true > 0, y_true * (lyt - ylog), 0.0)
    klr = jnp.sum(kl_el, axis=1)
    act = (seg[:, 0] != 0).astype(jnp.float32)
    acc_ref[0] += jnp.sum(klr * act)
    acc_ref[1] += jnp.sum(act)

    @pl.when(i == pl.num_programs(0) - 1)
    def _fin():
        loss_ref[0, 0] = acc_ref[0] / (acc_ref[1] + 1e-9)


def _loss(logits2, samples2, seg_col):
    bm = 512
    return pl.pallas_call(
        _loss_body,
        grid=(_BN // bm,),
        in_specs=[
            pl.BlockSpec((bm, _S), lambda i: (i, 0)),
            pl.BlockSpec((bm, _S), lambda i: (i, 0)),
            pl.BlockSpec((bm, 1), lambda i: (i, 0)),
        ],
        out_specs=pl.BlockSpec(memory_space=pltpu.SMEM),
        out_shape=jax.ShapeDtypeStruct((1, 1), jnp.float32),
        scratch_shapes=[pltpu.SMEM((2,), jnp.float32)],
    )(logits2, samples2, seg_col)


def _phase(d):
    return divmod(d - 1, 2)


def kernel(img, w_stem1, w_stem2, wq1, wq2, wk1, wk2, segment_target):
    xn = _normalize(img)

    xr = xn.reshape(_B, 3, 128, 2, 128, 2)
    xp1 = jnp.pad(xr, ((0, 0), (0, 0), (1, 0), (0, 0), (1, 0), (0, 0)))
    tap_slices = []
    for dy in range(3):
        u, p = _phase(dy)
        for dx in range(3):
            v, q = _phase(dx)
            for c in range(3):
                tap_slices.append(
                    xp1[:, c, 1 + u:129 + u, p, 1 + v:129 + v, q])
    patches1 = jnp.stack(tap_slices).reshape(27, _B * 128 * 128)
    w1 = jnp.transpose(w_stem1, (2, 3, 1, 0)).reshape(27, 64)
    f1 = _conv1(patches1, w1)

    f1r = f1.reshape(_B, 64, 2, 64, 2, 64)
    fp2 = jnp.pad(f1r, ((0, 0), (1, 0), (0, 0), (1, 0), (0, 0), (0, 0)))
    tap2 = []
    for dy in range(3):
        u, p = _phase(dy)
        for dx in range(3):
            v, q = _phase(dx)
            tap2.append(fp2[:, 1 + u:65 + u, p, 1 + v:65 + v, q, :])
    p2 = jnp.stack(tap2).reshape(9, _BN, 64)
    w2s = jnp.transpose(w_stem2, (2, 3, 1, 0)).reshape(9, 64, 256)
    w2s = w2s.astype(jnp.bfloat16)
    wq1m = jnp.transpose(wq1.reshape(64, 256), (1, 0)).astype(jnp.bfloat16)
    wq2m = jnp.transpose(wq2.reshape(32, 64), (1, 0)).astype(jnp.bfloat16)
    wk1m = jnp.transpose(wk1.reshape(64, 256), (1, 0)).astype(jnp.bfloat16)
    wk2m = jnp.transpose(wk2.reshape(32, 64), (1, 0)).astype(jnp.bfloat16)
    qf, kf = _heads(p2, w2s, wq1m, wq2m, wk1m, wk2m)

    qf3 = qf.reshape(_B, _N, 32)
    kT = jnp.transpose(kf.reshape(_B, _N, 32), (0, 2, 1))
    scores = _scores(qf3, kT)

    scoresw = lax.bitcast_convert_type(
        scores.reshape(_BN, _N // 2, 2), jnp.int32)
    indsw = _sample_inds_packed()
    seg = segment_target[:, 0, ::4, ::4].reshape(_B, _N)
    logits2, sampw = _sc_gather(scoresw, indsw, seg)
    samples2 = lax.bitcast_convert_type(sampw, jnp.int16).reshape(_BN, _S)

    loss = _loss(logits2, samples2, seg.reshape(_BN, 1))

    return logits2.reshape(_B, _N, _S), loss.reshape(())


# speedup vs baseline: 1.9821x; 1.9821x over previous
import functools

import numpy as np
import jax
import jax.numpy as jnp
from jax import lax
from jax.experimental import pallas as pl
from jax.experimental.pallas import tpu as pltpu
from jax.experimental.pallas import tpu_sc as plsc

_KWIN = 25
_S = 1024
_B = 2
_HW = 64
_N = _HW * _HW
_BN = _B * _N
_SCALE = 1.0 / np.sqrt(32.0)


def _local_indices_np(H, W, K):
    r = K // 2
    d = np.arange(-r, r + 1)
    ci = np.clip(np.arange(H)[:, None] + d[None, :], 0, H - 1)
    cj = np.clip(np.arange(W)[:, None] + d[None, :], 0, W - 1)
    idx = (ci[:, None, :, None] * W + cj[None, :, None, :]).reshape(H * W, K * K)
    return idx.astype(np.int32)


@functools.lru_cache(maxsize=1)
def _sample_inds_packed():
    local = np.broadcast_to(_local_indices_np(_HW, _HW, _KWIN)[None],
                            (_B, _N, _KWIN * _KWIN))
    n_rand = _S - _KWIN * _KWIN
    with jax.ensure_compile_time_eval():
        rnd = np.asarray(
            jax.random.randint(jax.random.key(42), (_B, _N, n_rand),
                               0, _N, dtype=jnp.int32))
    inds = np.concatenate([local, rnd], axis=-1).astype(np.int32)
    iw = inds.reshape(_BN, _S // 32, 16, 2)
    words = iw[..., 0] | (iw[..., 1] << 16)
    return jnp.asarray(words.reshape(_BN, _S // 2))


def _norm_body(x_ref, o_ref):
    x = x_ref[...]
    row = lax.broadcasted_iota(jnp.int32, x.shape, 0) % 3
    mean = jnp.where(row == 0, 123.675, jnp.where(row == 1, 116.28, 103.53))
    inv = jnp.where(row == 0, 1.0 / 58.395,
                    jnp.where(row == 1, 1.0 / 57.12, 1.0 / 57.375))
    o_ref[...] = (x - mean) * inv


def _normalize(img):
    x = img.reshape(_B * 3, 256 * 256)
    out = pl.pallas_call(
        _norm_body,
        out_shape=jax.ShapeDtypeStruct(x.shape, jnp.float32),
    )(x)
    return out.reshape(_B, 3, 256, 256)


def _conv1_body(x_ref, w_ref, o_ref):
    o_ref[...] = jax.nn.relu(
        lax.dot_general(x_ref[...], w_ref[...], (((0,), (0,)), ((), ())),
                        preferred_element_type=jnp.float32)).astype(jnp.bfloat16)


def _conv1(patches, w1):
    bm = 4096
    m = patches.shape[1]
    return pl.pallas_call(
        _conv1_body,
        grid=(m // bm,),
        in_specs=[
            pl.BlockSpec((27, bm), lambda i: (0, i)),
            pl.BlockSpec((27, 64), lambda i: (0, 0)),
        ],
        out_specs=pl.BlockSpec((bm, 64), lambda i: (i, 0)),
        out_shape=jax.ShapeDtypeStruct((m, 64), jnp.bfloat16),
    )(patches, w1)


def _heads_body(x_ref, w2_ref, wq1_ref, wq2_ref, wk1_ref, wk2_ref,
                q_ref, k_ref):
    acc = jnp.dot(x_ref[0], w2_ref[0], preferred_element_type=jnp.float32)
    for t in range(1, 9):
        acc += jnp.dot(x_ref[t], w2_ref[t], preferred_element_type=jnp.float32)
    f2 = jax.nn.relu(acc)
    qh = jax.nn.relu(
        jnp.dot(f2, wq1_ref[...], preferred_element_type=jnp.float32))
    q_ref[...] = jnp.dot(qh, wq2_ref[...],
                         preferred_element_type=jnp.float32).astype(jnp.bfloat16)
    kh = jax.nn.relu(
        jnp.dot(f2, wk1_ref[...], preferred_element_type=jnp.float32))
    k_ref[...] = jnp.dot(kh, wk2_ref[...],
                         preferred_element_type=jnp.float32).astype(jnp.bfloat16)


def _heads(p2, w2s, wq1m, wq2m, wk1m, wk2m):
    bm = 512
    qf, kf = pl.pallas_call(
        _heads_body,
        grid=(_BN // bm,),
        in_specs=[
            pl.BlockSpec((9, bm, 64), lambda i: (0, i, 0)),
            pl.BlockSpec((9, 64, 256), lambda i: (0, 0, 0)),
            pl.BlockSpec((256, 64), lambda i: (0, 0)),
            pl.BlockSpec((64, 32), lambda i: (0, 0)),
            pl.BlockSpec((256, 64), lambda i: (0, 0)),
            pl.BlockSpec((64, 32), lambda i: (0, 0)),
        ],
        out_specs=[
            pl.BlockSpec((bm, 32), lambda i: (i, 0)),
            pl.BlockSpec((bm, 32), lambda i: (i, 0)),
        ],
        out_shape=[
            jax.ShapeDtypeStruct((_BN, 32), jnp.bfloat16),
            jax.ShapeDtypeStruct((_BN, 32), jnp.bfloat16),
        ],
    )(p2, w2s, wq1m, wq2m, wk1m, wk2m)
    return qf, kf


def _scores_body(q_ref, kt_ref, o_ref):
    s = jnp.dot(q_ref[0], kt_ref[0], preferred_element_type=jnp.float32)
    sb = (s * _SCALE).astype(jnp.bfloat16)
    lo = lax.bitcast_convert_type(sb[:, :_N // 2], jnp.int16).astype(jnp.int32)
    hi = lax.bitcast_convert_type(sb[:, _N // 2:], jnp.int16).astype(jnp.int32)
    o_ref[0] = jnp.bitwise_or(jnp.bitwise_and(lo, 0xFFFF),
                              lax.shift_left(hi, 16))


def _scores(qf3, kT):
    bm = 256
    return pl.pallas_call(
        _scores_body,
        grid=(_B, _N // bm),
        in_specs=[
            pl.BlockSpec((1, bm, 32), lambda b, i: (b, i, 0)),
            pl.BlockSpec((1, 32, _N), lambda b, i: (b, 0, 0)),
        ],
        out_specs=pl.BlockSpec((1, bm, _N // 2), lambda b, i: (b, i, 0)),
        out_shape=jax.ShapeDtypeStruct((_B, _N, _N // 2), jnp.int32),
    )(qf3, kT)


def _sc_gather(scoresw, indsw, seg):
    NW = 32
    RPW = _BN // NW
    NCH = _S // 32
    mesh = plsc.VectorSubcoreMesh(core_axis_name="c", subcore_axis_name="s")

    @functools.partial(
        pl.kernel,
        out_type=[
            jax.ShapeDtypeStruct((_BN, _S), jnp.float32),
            jax.ShapeDtypeStruct((_BN, _S), jnp.int32),
        ],
        mesh=mesh,
        compiler_params=pltpu.CompilerParams(needs_layout_passes=False),
        scratch_types=[
            pltpu.VMEM((_N // 2,), jnp.int32),
            pltpu.VMEM((_N // 2,), jnp.int32),
            pltpu.VMEM((_S // 2,), jnp.int32),
            pltpu.VMEM((_S // 2,), jnp.int32),
            pltpu.VMEM((_S,), jnp.float32),
            pltpu.VMEM((_S,), jnp.float32),
            pltpu.VMEM((_S,), jnp.int32),
            pltpu.VMEM((_S,), jnp.int32),
            pltpu.VMEM((_N,), jnp.int32),
        ] + [pltpu.SemaphoreType.DMA] * 8,
    )
    def k(scores_hbm, inds_hbm, seg_hbm, lg_hbm, sm_hbm,
          srow0, srow1, idx0, idx1, out0, out1, samp0, samp1, segv,
          si0, si1, ii0, ii1, lo0, lo1, so0, so1):
        srows, idxs, outs, samps = (srow0, srow1), (idx0, idx1), (out0, out1), (samp0, samp1)
        sis, iis, los, sos = (si0, si1), (ii0, ii1), (lo0, lo1), (so0, so1)
        wid = lax.axis_index("s") * 2 + lax.axis_index("c")
        base = wid * RPW
        b = wid // (NW // _B)
        pltpu.sync_copy(seg_hbm.at[b], segv)
        pltpu.async_copy(scores_hbm.at[base], srow0, si0)
        pltpu.async_copy(inds_hbm.at[base], idx0, ii0)
        iota = lax.iota(jnp.int32, 16)

        def pair_body(pr, carry):
            for s in range(2):
                r = pr * 2 + s
                row = base + r

                @pl.when(r + 1 < RPW)
                def _prefetch():
                    pltpu.async_copy(scores_hbm.at[row + 1], srows[1 - s],
                                     sis[1 - s])
                    pltpu.async_copy(inds_hbm.at[row + 1], idxs[1 - s],
                                     iis[1 - s])

                pltpu.make_async_copy(scores_hbm.at[row], srows[s], sis[s]).wait()
                pltpu.make_async_copy(inds_hbm.at[row], idxs[s], iis[s]).wait()

                @pl.when(r >= 2)
                def _drain():
                    pltpu.make_async_copy(outs[s], lg_hbm.at[row - 2],
                                          los[s]).wait()
                    pltpu.make_async_copy(samps[s], sm_hbm.at[row - 2],
                                          sos[s]).wait()

                for c in range(NCH):
                    w = idxs[s][pl.ds(c * 16, 16)]
                    ia = jnp.bitwise_and(w, 0xFFFF)
                    ib = lax.shift_right_logical(w, 16)
                    wa = plsc.load_gather(srows[s],
                                          [jnp.bitwise_and(ia, _N // 2 - 1)])
                    wb = plsc.load_gather(srows[s],
                                          [jnp.bitwise_and(ib, _N // 2 - 1)])
                    sha = lax.shift_left(lax.shift_right_logical(ia, 11), 4)
                    shb = lax.shift_left(lax.shift_right_logical(ib, 11), 4)
                    va = plsc.bitcast(
                        lax.shift_left(lax.shift_right_logical(wa, sha), 16),
                        jnp.float32)
                    vb = plsc.bitcast(
                        lax.shift_left(lax.shift_right_logical(wb, shb), 16),
                        jnp.float32)
                    posa = c * 32 + iota * 2
                    plsc.store_scatter(outs[s], [posa], va)
                    plsc.store_scatter(outs[s], [posa + 1], vb)
                    sa = plsc.load_gather(segv, [ia])
                    sb = plsc.load_gather(segv, [ib])
                    plsc.store_scatter(samps[s], [posa], sa)
                    plsc.store_scatter(samps[s], [posa + 1], sb)

                pltpu.async_copy(outs[s], lg_hbm.at[row], los[s])
                pltpu.async_copy(samps[s], sm_hbm.at[row], sos[s])
            return carry

        lax.fori_loop(0, RPW // 2, pair_body, 0)
        pltpu.make_async_copy(out0, lg_hbm.at[base + RPW - 2], lo0).wait()
        pltpu.make_async_copy(samp0, sm_hbm.at[base + RPW - 2], so0).wait()
        pltpu.make_async_copy(out1, lg_hbm.at[base + RPW - 1], lo1).wait()
        pltpu.make_async_copy(samp1, sm_hbm.at[base + RPW - 1], so1).wait()

    return k(scoresw, indsw, seg)


def _loss_body(l_ref, s_ref, g_ref, loss_ref, acc_ref):
    i = pl.program_id(0)

    @pl.when(i == 0)
    def _init():
        acc_ref[0] = 0.0
        acc_ref[1] = 0.0

    l = l_ref[...]
    m = jnp.max(l, axis=1, keepdims=True)
    ex = jnp.exp(l - m)
    den = jnp.sum(ex, axis=1, keepdims=True)
    y_pred = ex / (den + 1e-12)
    ylog = jnp.log(jnp.maximum(y_pred, 1e-8))
    seg = g_ref[...]
    t = (s_ref[...].astype(jnp.int32) == seg).astype(jnp.float32)
    tsum = jnp.sum(t, axis=1, keepdims=True)
    y_true = t / (tsum + 1e-9)
    lyt = jnp.log(jnp.maximum(y_true, 1e-30))
    kl_el = jnp.where(y_true > 0, y_true * (lyt - ylog), 0.0)
    klr = jnp.sum(kl_el, axis=1)
    act = (seg[:, 0] != 0).astype(jnp.float32)
    acc_ref[0] += jnp.sum(klr * act)
    acc_ref[1] += jnp.sum(act)

    @pl.when(i == pl.num_programs(0) - 1)
    def _fin():
        loss_ref[0, 0] = acc_ref[0] / (acc_ref[1] + 1e-9)


def _loss(logits2, samples2, seg_col):
    bm = 512
    return pl.pallas_call(
        _loss_body,
        grid=(_BN // bm,),
        in_specs=[
            pl.BlockSpec((bm, _S), lambda i: (i, 0)),
            pl.BlockSpec((bm, _S), lambda i: (i, 0)),
            pl.BlockSpec((bm, 1), lambda i: (i, 0)),
        ],
        out_specs=pl.BlockSpec(memory_space=pltpu.SMEM),
        out_shape=jax.ShapeDtypeStruct((1, 1), jnp.float32),
        scratch_shapes=[pltpu.SMEM((2,), jnp.float32)],
    )(logits2, samples2, seg_col)


def _phase(d):
    return divmod(d - 1, 2)


def kernel(img, w_stem1, w_stem2, wq1, wq2, wk1, wk2, segment_target):
    xn = _normalize(img)

    xr = xn.reshape(_B, 3, 128, 2, 128, 2)
    xp1 = jnp.pad(xr, ((0, 0), (0, 0), (1, 0), (0, 0), (1, 0), (0, 0)))
    tap_slices = []
    for dy in range(3):
        u, p = _phase(dy)
        for dx in range(3):
            v, q = _phase(dx)
            for c in range(3):
                tap_slices.append(
                    xp1[:, c, 1 + u:129 + u, p, 1 + v:129 + v, q])
    patches1 = jnp.stack(tap_slices).reshape(27, _B * 128 * 128)
    w1 = jnp.transpose(w_stem1, (2, 3, 1, 0)).reshape(27, 64)
    f1 = _conv1(patches1, w1)

    f1r = f1.reshape(_B, 64, 2, 64, 2, 64)
    fp2 = jnp.pad(f1r, ((0, 0), (1, 0), (0, 0), (1, 0), (0, 0), (0, 0)))
    tap2 = []
    for dy in range(3):
        u, p = _phase(dy)
        for dx in range(3):
            v, q = _phase(dx)
            tap2.append(fp2[:, 1 + u:65 + u, p, 1 + v:65 + v, q, :])
    p2 = jnp.stack(tap2).reshape(9, _BN, 64)
    w2s = jnp.transpose(w_stem2, (2, 3, 1, 0)).reshape(9, 64, 256)
    w2s = w2s.astype(jnp.bfloat16)
    wq1m = jnp.transpose(wq1.reshape(64, 256), (1, 0)).astype(jnp.bfloat16)
    wq2m = jnp.transpose(wq2.reshape(32, 64), (1, 0)).astype(jnp.bfloat16)
    wk1m = jnp.transpose(wk1.reshape(64, 256), (1, 0)).astype(jnp.bfloat16)
    wk2m = jnp.transpose(wk2.reshape(32, 64), (1, 0)).astype(jnp.bfloat16)
    qf, kf = _heads(p2, w2s, wq1m, wq2m, wk1m, wk2m)

    qf3 = qf.reshape(_B, _N, 32)
    kT = jnp.transpose(kf.reshape(_B, _N, 32), (0, 2, 1))
    scoresw = _scores(qf3, kT).reshape(_BN, _N // 2)

    indsw = _sample_inds_packed()
    seg = segment_target[:, 0, ::4, ::4].reshape(_B, _N)
    logits2, samples2 = _sc_gather(scoresw, indsw, seg)

    loss = _loss(logits2, samples2, seg.reshape(_BN, 1))

    return logits2.reshape(_B, _N, _S), loss.reshape(())


# speedup vs baseline: 2.0596x; 1.0391x over previous
import functools

import numpy as np
import jax
import jax.numpy as jnp
from jax import lax
from jax.experimental import pallas as pl
from jax.experimental.pallas import tpu as pltpu
from jax.experimental.pallas import tpu_sc as plsc

_KWIN = 25
_S = 1024
_B = 2
_HW = 64
_N = _HW * _HW
_BN = _B * _N
_SCALE = 1.0 / np.sqrt(32.0)


def _local_indices_np(H, W, K):
    r = K // 2
    d = np.arange(-r, r + 1)
    ci = np.clip(np.arange(H)[:, None] + d[None, :], 0, H - 1)
    cj = np.clip(np.arange(W)[:, None] + d[None, :], 0, W - 1)
    idx = (ci[:, None, :, None] * W + cj[None, :, None, :]).reshape(H * W, K * K)
    return idx.astype(np.int32)


@functools.lru_cache(maxsize=1)
def _sample_inds_packed():
    local = np.broadcast_to(_local_indices_np(_HW, _HW, _KWIN)[None],
                            (_B, _N, _KWIN * _KWIN))
    n_rand = _S - _KWIN * _KWIN
    with jax.ensure_compile_time_eval():
        rnd = np.asarray(
            jax.random.randint(jax.random.key(42), (_B, _N, n_rand),
                               0, _N, dtype=jnp.int32))
    inds = np.concatenate([local, rnd], axis=-1).astype(np.int32)
    iw = inds.reshape(_BN, _S // 32, 2, 16)
    words = iw[:, :, 0, :] | (iw[:, :, 1, :] << 16)
    return jnp.asarray(words.reshape(_BN, _S // 2))


def _norm_body(x_ref, o_ref):
    x = x_ref[...]
    row = lax.broadcasted_iota(jnp.int32, x.shape, 0) % 3
    mean = jnp.where(row == 0, 123.675, jnp.where(row == 1, 116.28, 103.53))
    inv = jnp.where(row == 0, 1.0 / 58.395,
                    jnp.where(row == 1, 1.0 / 57.12, 1.0 / 57.375))
    o_ref[...] = (x - mean) * inv


def _normalize(img):
    x = img.reshape(_B * 3, 256 * 256)
    out = pl.pallas_call(
        _norm_body,
        out_shape=jax.ShapeDtypeStruct(x.shape, jnp.float32),
    )(x)
    return out.reshape(_B, 3, 256, 256)


def _conv1_body(x_ref, w_ref, o_ref):
    o_ref[...] = jax.nn.relu(
        lax.dot_general(x_ref[...], w_ref[...], (((0,), (0,)), ((), ())),
                        preferred_element_type=jnp.float32)).astype(jnp.bfloat16)


def _conv1(patches, w1):
    bm = 4096
    m = patches.shape[1]
    return pl.pallas_call(
        _conv1_body,
        grid=(m // bm,),
        in_specs=[
            pl.BlockSpec((27, bm), lambda i: (0, i)),
            pl.BlockSpec((27, 64), lambda i: (0, 0)),
        ],
        out_specs=pl.BlockSpec((bm, 64), lambda i: (i, 0)),
        out_shape=jax.ShapeDtypeStruct((m, 64), jnp.bfloat16),
    )(patches, w1)


def _heads_body(x_ref, w2_ref, wq1_ref, wq2_ref, wk1_ref, wk2_ref,
                q_ref, k_ref):
    acc = jnp.dot(x_ref[0], w2_ref[0], preferred_element_type=jnp.float32)
    for t in range(1, 9):
        acc += jnp.dot(x_ref[t], w2_ref[t], preferred_element_type=jnp.float32)
    f2 = jax.nn.relu(acc)
    qh = jax.nn.relu(
        jnp.dot(f2, wq1_ref[...], preferred_element_type=jnp.float32))
    q_ref[...] = jnp.dot(qh, wq2_ref[...],
                         preferred_element_type=jnp.float32).astype(jnp.bfloat16)
    kh = jax.nn.relu(
        jnp.dot(f2, wk1_ref[...], preferred_element_type=jnp.float32))
    k_ref[...] = jnp.dot(kh, wk2_ref[...],
                         preferred_element_type=jnp.float32).astype(jnp.bfloat16)


def _heads(p2, w2s, wq1m, wq2m, wk1m, wk2m):
    bm = 512
    qf, kf = pl.pallas_call(
        _heads_body,
        grid=(_BN // bm,),
        in_specs=[
            pl.BlockSpec((9, bm, 64), lambda i: (0, i, 0)),
            pl.BlockSpec((9, 64, 256), lambda i: (0, 0, 0)),
            pl.BlockSpec((256, 64), lambda i: (0, 0)),
            pl.BlockSpec((64, 32), lambda i: (0, 0)),
            pl.BlockSpec((256, 64), lambda i: (0, 0)),
            pl.BlockSpec((64, 32), lambda i: (0, 0)),
        ],
        out_specs=[
            pl.BlockSpec((bm, 32), lambda i: (i, 0)),
            pl.BlockSpec((bm, 32), lambda i: (i, 0)),
        ],
        out_shape=[
            jax.ShapeDtypeStruct((_BN, 32), jnp.bfloat16),
            jax.ShapeDtypeStruct((_BN, 32), jnp.bfloat16),
        ],
    )(p2, w2s, wq1m, wq2m, wk1m, wk2m)
    return qf, kf


def _scores_body(q_ref, kt_ref, o_ref):
    s = jnp.dot(q_ref[0], kt_ref[0], preferred_element_type=jnp.float32)
    sb = (s * _SCALE).astype(jnp.bfloat16)
    lo = lax.bitcast_convert_type(sb[:, :_N // 2], jnp.int16).astype(jnp.int32)
    hi = lax.bitcast_convert_type(sb[:, _N // 2:], jnp.int16).astype(jnp.int32)
    o_ref[0] = jnp.bitwise_or(jnp.bitwise_and(lo, 0xFFFF),
                              lax.shift_left(hi, 16))


def _scores(qf3, kT):
    bm = 256
    return pl.pallas_call(
        _scores_body,
        grid=(_B, _N // bm),
        in_specs=[
            pl.BlockSpec((1, bm, 32), lambda b, i: (b, i, 0)),
            pl.BlockSpec((1, 32, _N), lambda b, i: (b, 0, 0)),
        ],
        out_specs=pl.BlockSpec((1, bm, _N // 2), lambda b, i: (b, i, 0)),
        out_shape=jax.ShapeDtypeStruct((_B, _N, _N // 2), jnp.int32),
    )(qf3, kT)


def _sc_gather(scoresw, indsw, seg):
    NW = 32
    RPW = _BN // NW
    NCH = _S // 32
    mesh = plsc.VectorSubcoreMesh(core_axis_name="c", subcore_axis_name="s")

    @functools.partial(
        pl.kernel,
        out_type=[
            jax.ShapeDtypeStruct((_BN, _S), jnp.float32),
            jax.ShapeDtypeStruct((_BN, _S), jnp.int32),
        ],
        mesh=mesh,
        compiler_params=pltpu.CompilerParams(needs_layout_passes=False),
        scratch_types=[
            pltpu.VMEM((_N // 2,), jnp.int32),
            pltpu.VMEM((_N // 2,), jnp.int32),
            pltpu.VMEM((_S // 2,), jnp.int32),
            pltpu.VMEM((_S // 2,), jnp.int32),
            pltpu.VMEM((_S,), jnp.float32),
            pltpu.VMEM((_S,), jnp.float32),
            pltpu.VMEM((_S,), jnp.int32),
            pltpu.VMEM((_S,), jnp.int32),
            pltpu.VMEM((_N,), jnp.int32),
        ] + [pltpu.SemaphoreType.DMA] * 8,
    )
    def k(scores_hbm, inds_hbm, seg_hbm, lg_hbm, sm_hbm,
          srow0, srow1, idx0, idx1, out0, out1, samp0, samp1, segv,
          si0, si1, ii0, ii1, lo0, lo1, so0, so1):
        srows, idxs, outs, samps = (srow0, srow1), (idx0, idx1), (out0, out1), (samp0, samp1)
        sis, iis, los, sos = (si0, si1), (ii0, ii1), (lo0, lo1), (so0, so1)
        wid = lax.axis_index("s") * 2 + lax.axis_index("c")
        base = wid * RPW
        b = wid // (NW // _B)
        pltpu.sync_copy(seg_hbm.at[b], segv)
        pltpu.async_copy(scores_hbm.at[base], srow0, si0)
        pltpu.async_copy(inds_hbm.at[base], idx0, ii0)

        def pair_body(pr, carry):
            for s in range(2):
                r = pr * 2 + s
                row = base + r

                @pl.when(r + 1 < RPW)
                def _prefetch():
                    pltpu.async_copy(scores_hbm.at[row + 1], srows[1 - s],
                                     sis[1 - s])
                    pltpu.async_copy(inds_hbm.at[row + 1], idxs[1 - s],
                                     iis[1 - s])

                pltpu.make_async_copy(scores_hbm.at[row], srows[s], sis[s]).wait()
                pltpu.make_async_copy(inds_hbm.at[row], idxs[s], iis[s]).wait()

                @pl.when(r >= 2)
                def _drain():
                    pltpu.make_async_copy(outs[s], lg_hbm.at[row - 2],
                                          los[s]).wait()
                    pltpu.make_async_copy(samps[s], sm_hbm.at[row - 2],
                                          sos[s]).wait()

                for c in range(NCH):
                    w = idxs[s][pl.ds(c * 16, 16)]
                    ia = jnp.bitwise_and(w, 0xFFFF)
                    ib = lax.shift_right_logical(w, 16)
                    wa = plsc.load_gather(srows[s],
                                          [jnp.bitwise_and(ia, _N // 2 - 1)])
                    wb = plsc.load_gather(srows[s],
                                          [jnp.bitwise_and(ib, _N // 2 - 1)])
                    sha = lax.shift_left(lax.shift_right_logical(ia, 11), 4)
                    shb = lax.shift_left(lax.shift_right_logical(ib, 11), 4)
                    va = plsc.bitcast(
                        lax.shift_left(lax.shift_right_logical(wa, sha), 16),
                        jnp.float32)
                    vb = plsc.bitcast(
                        lax.shift_left(lax.shift_right_logical(wb, shb), 16),
                        jnp.float32)
                    outs[s][pl.ds(c * 32, 16)] = va
                    outs[s][pl.ds(c * 32 + 16, 16)] = vb
                    sa = plsc.load_gather(segv, [ia])
                    sb = plsc.load_gather(segv, [ib])
                    samps[s][pl.ds(c * 32, 16)] = sa
                    samps[s][pl.ds(c * 32 + 16, 16)] = sb

                pltpu.async_copy(outs[s], lg_hbm.at[row], los[s])
                pltpu.async_copy(samps[s], sm_hbm.at[row], sos[s])
            return carry

        lax.fori_loop(0, RPW // 2, pair_body, 0)
        pltpu.make_async_copy(out0, lg_hbm.at[base + RPW - 2], lo0).wait()
        pltpu.make_async_copy(samp0, sm_hbm.at[base + RPW - 2], so0).wait()
        pltpu.make_async_copy(out1, lg_hbm.at[base + RPW - 1], lo1).wait()
        pltpu.make_async_copy(samp1, sm_hbm.at[base + RPW - 1], so1).wait()

    return k(scoresw, indsw, seg)


def _loss_body(l_ref, s_ref, g_ref, loss_ref, acc_ref):
    i = pl.program_id(0)

    @pl.when(i == 0)
    def _init():
        acc_ref[0] = 0.0
        acc_ref[1] = 0.0

    l = l_ref[...]
    m = jnp.max(l, axis=1, keepdims=True)
    ex = jnp.exp(l - m)
    den = jnp.sum(ex, axis=1, keepdims=True)
    y_pred = ex / (den + 1e-12)
    ylog = jnp.log(jnp.maximum(y_pred, 1e-8))
    seg = g_ref[...]
    t = (s_ref[...].astype(jnp.int32) == seg).astype(jnp.float32)
    tsum = jnp.sum(t, axis=1, keepdims=True)
    y_true = t / (tsum + 1e-9)
    lyt = jnp.log(jnp.maximum(y_true, 1e-30))
    kl_el = jnp.where(y_true > 0, y_true * (lyt - ylog), 0.0)
    klr = jnp.sum(kl_el, axis=1)
    act = (seg[:, 0] != 0).astype(jnp.float32)
    acc_ref[0] += jnp.sum(klr * act)
    acc_ref[1] += jnp.sum(act)

    @pl.when(i == pl.num_programs(0) - 1)
    def _fin():
        loss_ref[0, 0] = acc_ref[0] / (acc_ref[1] + 1e-9)


def _loss(logits2, samples2, seg_col):
    bm = 512
    return pl.pallas_call(
        _loss_body,
        grid=(_BN // bm,),
        in_specs=[
            pl.BlockSpec((bm, _S), lambda i: (i, 0)),
            pl.BlockSpec((bm, _S), lambda i: (i, 0)),
            pl.BlockSpec((bm, 1), lambda i: (i, 0)),
        ],
        out_specs=pl.BlockSpec(memory_space=pltpu.SMEM),
        out_shape=jax.ShapeDtypeStruct((1, 1), jnp.float32),
        scratch_shapes=[pltpu.SMEM((2,), jnp.float32)],
    )(logits2, samples2, seg_col)


def _phase(d):
    return divmod(d - 1, 2)


def kernel(img, w_stem1, w_stem2, wq1, wq2, wk1, wk2, segment_target):
    xn = _normalize(img)

    xr = xn.reshape(_B, 3, 128, 2, 128, 2)
    xp1 = jnp.pad(xr, ((0, 0), (0, 0), (1, 0), (0, 0), (1, 0), (0, 0)))
    tap_slices = []
    for dy in range(3):
        u, p = _phase(dy)
        for dx in range(3):
            v, q = _phase(dx)
            for c in range(3):
                tap_slices.append(
                    xp1[:, c, 1 + u:129 + u, p, 1 + v:129 + v, q])
    patches1 = jnp.stack(tap_slices).reshape(27, _B * 128 * 128)
    w1 = jnp.transpose(w_stem1, (2, 3, 1, 0)).reshape(27, 64)
    f1 = _conv1(patches1, w1)

    f1r = f1.reshape(_B, 64, 2, 64, 2, 64)
    fp2 = jnp.pad(f1r, ((0, 0), (1, 0), (0, 0), (1, 0), (0, 0), (0, 0)))
    tap2 = []
    for dy in range(3):
        u, p = _phase(dy)
        for dx in range(3):
            v, q = _phase(dx)
            tap2.append(fp2[:, 1 + u:65 + u, p, 1 + v:65 + v, q, :])
    p2 = jnp.stack(tap2).reshape(9, _BN, 64)
    w2s = jnp.transpose(w_stem2, (2, 3, 1, 0)).reshape(9, 64, 256)
    w2s = w2s.astype(jnp.bfloat16)
    wq1m = jnp.transpose(wq1.reshape(64, 256), (1, 0)).astype(jnp.bfloat16)
    wq2m = jnp.transpose(wq2.reshape(32, 64), (1, 0)).astype(jnp.bfloat16)
    wk1m = jnp.transpose(wk1.reshape(64, 256), (1, 0)).astype(jnp.bfloat16)
    wk2m = jnp.transpose(wk2.reshape(32, 64), (1, 0)).astype(jnp.bfloat16)
    qf, kf = _heads(p2, w2s, wq1m, wq2m, wk1m, wk2m)

    qf3 = qf.reshape(_B, _N, 32)
    kT = jnp.transpose(kf.reshape(_B, _N, 32), (0, 2, 1))
    scoresw = _scores(qf3, kT).reshape(_BN, _N // 2)

    indsw = _sample_inds_packed()
    seg = segment_target[:, 0, ::4, ::4].reshape(_B, _N)
    logits2, samples2 = _sc_gather(scoresw, indsw, seg)

    loss = _loss(logits2, samples2, seg.reshape(_BN, 1))

    return logits2.reshape(_B, _N, _S), loss.reshape(())
